# baseline (speedup 1.0000x reference)
import functools
import jax
import jax.numpy as jnp
from jax import lax
from jax.experimental import pallas as pl
from jax.experimental.pallas import tpu as pltpu
from jax.experimental.pallas import tpu_sc as plsc

B = 4096
D = 1024
E = 8
SYM = 128
HOPS = 4
TB = 256
NTB = B // TB
NG = 24
G = NG * TB
NGX = NG + NTB
BIGN = G + B

NW = 32
CHG = 96
CHM = 64


def _stage_a_body(x_ref, wp_ref, bp_ref, ws_ref, wrz_ref, wrs_ref,
                  z_ref, sym_ref, prog_ref, eff_ref, cnt_ref, acc):
    i = pl.program_id(0)
    x = x_ref[...]
    z = jnp.dot(x, wp_ref[...], preferred_element_type=jnp.float32) + bp_ref[...]
    z_ref[...] = z
    sym = jnp.tanh(jnp.dot(z, ws_ref[...], preferred_element_type=jnp.float32))
    sym_ref[...] = sym
    msym = jnp.mean(sym.reshape(TB, E, SYM), axis=1)
    logits = (jnp.dot(z, wrz_ref[...], preferred_element_type=jnp.float32)
              + jnp.dot(msym, wrs_ref[...], preferred_element_type=jnp.float32))
    lg = logits.reshape(TB, HOPS, E + 1)
    mx = jnp.max(lg, axis=-1, keepdims=True)
    k_iota = jax.lax.broadcasted_iota(jnp.int32, (TB, HOPS, E + 1), 2)
    idx = jnp.min(jnp.where(lg >= mx, k_iota, E + 1), axis=-1).astype(jnp.int32)
    prog_ref[...] = idx
    active = jnp.ones((TB, 1), dtype=jnp.bool_)
    effs = []
    for h in range(HOPS):
        ph = idx[:, h:h + 1]
        ok = active & (ph != E)
        effs.append(jnp.where(ok, ph, E).astype(jnp.int32))
        active = ok
    eff = jnp.concatenate(effs, axis=1)
    eff_ref[...] = eff
    e_iota = jax.lax.broadcasted_iota(jnp.int32, (TB, HOPS, E), 2)
    oh = (eff[:, :, None] == e_iota).astype(jnp.float32)
    c = jnp.sum(oh, axis=0)

    @pl.when(i == 0)
    def _():
        acc[...] = jnp.zeros_like(acc)

    acc[...] += c

    @pl.when(i == NTB - 1)
    def _():
        cnt_ref[...] = acc[...].astype(jnp.int32)


def _stage_a(x, Wp, bp2, Ws2, Wrz, Wrs):
    return pl.pallas_call(
        _stage_a_body,
        grid=(NTB,),
        in_specs=[
            pl.BlockSpec((TB, D), lambda i: (i, 0)),
            pl.BlockSpec((D, D), lambda i: (0, 0)),
            pl.BlockSpec((1, D), lambda i: (0, 0)),
            pl.BlockSpec((D, E * SYM), lambda i: (0, 0)),
            pl.BlockSpec((D, HOPS * (E + 1)), lambda i: (0, 0)),
            pl.BlockSpec((SYM, HOPS * (E + 1)), lambda i: (0, 0)),
        ],
        out_specs=[
            pl.BlockSpec((TB, D), lambda i: (i, 0)),
            pl.BlockSpec((TB, E * SYM), lambda i: (i, 0)),
            pl.BlockSpec((TB, HOPS), lambda i: (i, 0)),
            pl.BlockSpec((TB, HOPS), lambda i: (i, 0)),
            pl.BlockSpec((HOPS, E), lambda i: (0, 0)),
        ],
        out_shape=[
            jax.ShapeDtypeStruct((B, D), jnp.float32),
            jax.ShapeDtypeStruct((B, E * SYM), jnp.float32),
            jax.ShapeDtypeStruct((B, HOPS), jnp.int32),
            jax.ShapeDtypeStruct((B, HOPS), jnp.int32),
            jax.ShapeDtypeStruct((HOPS, E), jnp.int32),
        ],
        scratch_shapes=[pltpu.VMEM((HOPS, E), jnp.float32)],
    )(x, Wp, bp2, Ws2, Wrz, Wrs)


def _sched_body(cnt_ref, base_ref, be_ref):
    c = cnt_ref[...].astype(jnp.float32)
    nb = jnp.ceil(c / TB)
    e1 = jax.lax.broadcasted_iota(jnp.int32, (E, E), 0)
    e2 = jax.lax.broadcasted_iota(jnp.int32, (E, E), 1)
    mexc = (e1 < e2).astype(jnp.float32)
    minc = (e1 <= e2).astype(jnp.float32)
    excl = jnp.dot(nb, mexc, preferred_element_type=jnp.float32)
    incl = jnp.dot(nb, minc, preferred_element_type=jnp.float32)
    base_ref[...] = (TB * excl).astype(jnp.int32)
    g_iota = jax.lax.broadcasted_iota(jnp.int32, (HOPS, NG, E), 1)
    be = jnp.sum((incl.astype(jnp.int32)[:, None, :] <= g_iota).astype(jnp.int32),
                 axis=2)
    be_ref[...] = jnp.minimum(be, E - 1)


def _sched(cnt):
    return pl.pallas_call(
        _sched_body,
        out_shape=[
            jax.ShapeDtypeStruct((HOPS, E), jnp.int32),
            jax.ShapeDtypeStruct((HOPS, NG), jnp.int32),
        ],
    )(cnt)


def _pos_body(eff_ref, base_ref, pos_ref, mix_ref, carry):
    i = pl.program_id(0)

    @pl.when(i == 0)
    def _():
        carry[...] = jnp.zeros_like(carry)

    eff = eff_ref[...]
    base = base_ref[...].astype(jnp.float32)
    run = carry[...]
    r1 = jax.lax.broadcasted_iota(jnp.int32, (TB, TB), 0)
    c1 = jax.lax.broadcasted_iota(jnp.int32, (TB, TB), 1)
    ltri = (c1 < r1).astype(jnp.float32)
    gtok = (i * TB + jax.lax.broadcasted_iota(jnp.int32, (TB, 1), 0)
            ).astype(jnp.float32)
    pcols, mcols, newrun = [], [], []
    for h in range(HOPS):
        e_iota = jax.lax.broadcasted_iota(jnp.int32, (TB, E), 1)
        oh = (eff[:, h:h + 1] == e_iota).astype(jnp.float32)
        rank = jnp.dot(ltri, oh, preferred_element_type=jnp.float32)
        offs = base[h:h + 1, :] + run[h:h + 1, :]
        p = jnp.sum(oh * (rank + offs), axis=1, keepdims=True)
        is_act = jnp.sum(oh, axis=1, keepdims=True) > 0.0
        pcols.append(jnp.where(is_act, p, -1.0))
        mcols.append(jnp.where(is_act, p, G + gtok))
        newrun.append(run[h:h + 1, :] + jnp.sum(oh, axis=0, keepdims=True))
    pos_ref[...] = jnp.concatenate(pcols, axis=1).astype(jnp.int32)
    mix_ref[...] = jnp.concatenate(mcols, axis=1).astype(jnp.int32)
    carry[...] = jnp.concatenate(newrun, axis=0)


def _pos(eff, base):
    return pl.pallas_call(
        _pos_body,
        grid=(NTB,),
        in_specs=[
            pl.BlockSpec((TB, HOPS), lambda i: (i, 0)),
            pl.BlockSpec((HOPS, E), lambda i: (0, 0)),
        ],
        out_specs=[
            pl.BlockSpec((TB, HOPS), lambda i: (i, 0)),
            pl.BlockSpec((TB, HOPS), lambda i: (i, 0)),
        ],
        out_shape=[
            jax.ShapeDtypeStruct((B, HOPS), jnp.int32),
            jax.ShapeDtypeStruct((B, HOPS), jnp.int32),
        ],
        scratch_shapes=[pltpu.VMEM((HOPS, E), jnp.float32)],
    )(eff, base)


def _cids_body(pos_ref, out_ref):
    g = pl.program_id(0)
    pos = pos_ref[...]
    tok = jax.lax.broadcasted_iota(jnp.int32, (B, TB), 0)
    slot = g * TB + jax.lax.broadcasted_iota(jnp.int32, (B, TB), 1)
    big = jnp.int32(1 << 30)
    rows = []
    for h in range(HOPS):
        p = pos[:, h:h + 1]
        src = tok if h == 0 else pos[:, h - 1:h] + jnp.zeros_like(tok)
        cand = jnp.where(p == slot, src, big)
        row = jnp.min(cand, axis=0, keepdims=True)
        rows.append(jnp.where(row == big, 0, row))
    out_ref[...] = jnp.concatenate(rows, axis=0)


def _cids(pos):
    return pl.pallas_call(
        _cids_body,
        grid=(NG,),
        in_specs=[pl.BlockSpec((B, HOPS), lambda g: (0, 0))],
        out_specs=pl.BlockSpec((HOPS, TB), lambda g: (0, g)),
        out_shape=jax.ShapeDtypeStruct((HOPS, G), jnp.int32),
    )(pos)


def _expert_body(be_ref, g_ref, m_ref, wo_ref, bo_ref, o_ref):
    g = pl.program_id(0)

    @pl.when(g < NG)
    def _():
        o_ref[...] = jnp.tanh(
            jnp.dot(g_ref[...], wo_ref[0], preferred_element_type=jnp.float32)
            + bo_ref[0])

    @pl.when(g >= NG)
    def _():
        o_ref[...] = m_ref[...]


def _expert(be, gathered, merged_prev, Wo, bo3):
    grid_spec = pltpu.PrefetchScalarGridSpec(
        num_scalar_prefetch=1,
        grid=(NGX,),
        in_specs=[
            pl.BlockSpec((TB, D), lambda g, be: (jnp.minimum(g, NG - 1), 0)),
            pl.BlockSpec((TB, D), lambda g, be: (jnp.maximum(g - NG, 0), 0)),
            pl.BlockSpec((1, D, D),
                         lambda g, be: (be[jnp.minimum(g, NG - 1)], 0, 0)),
            pl.BlockSpec((1, 1, D),
                         lambda g, be: (be[jnp.minimum(g, NG - 1)], 0, 0)),
        ],
        out_specs=pl.BlockSpec((TB, D), lambda g, be: (g, 0)),
    )
    return pl.pallas_call(
        _expert_body,
        grid_spec=grid_spec,
        out_shape=jax.ShapeDtypeStruct((BIGN, D), jnp.float32),
    )(be, gathered, merged_prev, Wo, bo3)


def _sc_mesh():
    return plsc.VectorSubcoreMesh(core_axis_name="c", subcore_axis_name="s")


def _gather_sc(src, idx3, nch, ch):
    n_out = NW * nch * ch

    @functools.partial(
        pl.kernel,
        mesh=_sc_mesh(),
        out_type=jax.ShapeDtypeStruct((n_out, D), jnp.float32),
        scratch_types=[
            pltpu.VMEM((nch, ch), jnp.int32),
            pltpu.VMEM((ch, D), jnp.float32),
            pltpu.SemaphoreType.DMA,
        ],
    )
    def k(src_hbm, idx_hbm, out_hbm, idx_v, rows_v, sem):
        wid = lax.axis_index("s") * 2 + lax.axis_index("c")
        pltpu.sync_copy(idx_hbm.at[wid], idx_v)
        for j in range(nch):
            pltpu.async_copy(src_hbm.at[idx_v.at[j]], rows_v, sem).wait()
            pltpu.sync_copy(rows_v, out_hbm.at[pl.ds(wid * nch * ch + j * ch, ch)])

    return k(src, idx3)


def _gather_merge_sc(src, cid3, mix3):
    @functools.partial(
        pl.kernel,
        mesh=_sc_mesh(),
        out_type=[
            jax.ShapeDtypeStruct((G, D), jnp.float32),
            jax.ShapeDtypeStruct((B, D), jnp.float32),
        ],
        scratch_types=[
            pltpu.VMEM((2, CHG), jnp.int32),
            pltpu.VMEM((2, CHM), jnp.int32),
            pltpu.VMEM((CHG, D), jnp.float32),
            pltpu.SemaphoreType.DMA,
        ],
    )
    def k(src_hbm, cid_hbm, mix_hbm, gat_hbm, mrg_hbm, cid_v, mix_v, rows_v, sem):
        wid = lax.axis_index("s") * 2 + lax.axis_index("c")
        pltpu.sync_copy(cid_hbm.at[wid], cid_v)
        pltpu.sync_copy(mix_hbm.at[wid], mix_v)
        for j in range(2):
            pltpu.async_copy(src_hbm.at[cid_v.at[j]], rows_v, sem).wait()
            pltpu.sync_copy(rows_v, gat_hbm.at[pl.ds(wid * 2 * CHG + j * CHG, CHG)])
        for j in range(2):
            rv = rows_v.at[pl.ds(0, CHM)]
            pltpu.async_copy(src_hbm.at[mix_v.at[j]], rv, sem).wait()
            pltpu.sync_copy(rv, mrg_hbm.at[pl.ds(wid * 2 * CHM + j * CHM, CHM)])

    return k(src, cid3, mix3)


def kernel(x, Wp, bp, Wo, bo, Ws, Wr, max_ops):
    Ws2 = jnp.transpose(Ws, (1, 0, 2)).reshape(D, E * SYM)
    Wrz = jnp.transpose(Wr[:, :D, :], (1, 0, 2)).reshape(D, HOPS * (E + 1))
    Wrs = jnp.transpose(Wr[:, D:, :], (1, 0, 2)).reshape(SYM, HOPS * (E + 1))
    bp2 = bp.reshape(1, D)

    z, sym_flat, prog, eff, cnt = _stage_a(x, Wp, bp2, Ws2, Wrz, Wrs)
    basearr, be = _sched(cnt)
    pos, mix = _pos(eff, basearr)
    cids = _cids(pos)

    gathered = _gather_sc(z, cids[0].reshape(NW, 2, CHG), 2, CHG)
    bo3 = bo.reshape(E, 1, D)
    bigout = _expert(be[0], gathered, z, Wo, bo3)
    for h in range(1, HOPS):
        gathered, merged = _gather_merge_sc(
            bigout,
            cids[h].reshape(NW, 2, CHG),
            mix[:, h - 1].reshape(NW, 2, CHM),
        )
        bigout = _expert(be[h], gathered, merged, Wo, bo3)
    out = _gather_sc(bigout, mix[:, HOPS - 1].reshape(NW, 2, CHM), 2, CHM)

    return out, prog, sym_flat.reshape(B, E, SYM)


# speedup vs baseline: 1.1008x; 1.1008x over previous
import functools
import jax
import jax.numpy as jnp
from jax import lax
from jax.experimental import pallas as pl
from jax.experimental.pallas import tpu as pltpu
from jax.experimental.pallas import tpu_sc as plsc

B = 4096
D = 1024
E = 8
SYM = 128
HOPS = 4
TB = 256
NTB = B // TB
NG = 24
G = NG * TB
NGX = NG + NTB
BIGN = HOPS * G + B

NW = 32


def _stage_a_body(x_ref, wp_ref, bp_ref, ws_ref, wrz_ref, wrs_ref,
                  z_ref, sym_ref, prog_ref, eff_ref, cnt_ref, acc):
    i = pl.program_id(0)
    x = x_ref[...]
    z = jnp.dot(x, wp_ref[...], preferred_element_type=jnp.float32) + bp_ref[...]
    z_ref[...] = z
    sym = jnp.tanh(jnp.dot(z, ws_ref[...], preferred_element_type=jnp.float32))
    sym_ref[...] = sym
    msym = jnp.mean(sym.reshape(TB, E, SYM), axis=1)
    logits = (jnp.dot(z, wrz_ref[...], preferred_element_type=jnp.float32)
              + jnp.dot(msym, wrs_ref[...], preferred_element_type=jnp.float32))
    lg = logits.reshape(TB, HOPS, E + 1)
    mx = jnp.max(lg, axis=-1, keepdims=True)
    k_iota = jax.lax.broadcasted_iota(jnp.int32, (TB, HOPS, E + 1), 2)
    idx = jnp.min(jnp.where(lg >= mx, k_iota, E + 1), axis=-1).astype(jnp.int32)
    prog_ref[...] = idx
    active = jnp.ones((TB, 1), dtype=jnp.bool_)
    effs = []
    for h in range(HOPS):
        ph = idx[:, h:h + 1]
        ok = active & (ph != E)
        effs.append(jnp.where(ok, ph, E).astype(jnp.int32))
        active = ok
    eff = jnp.concatenate(effs, axis=1)
    eff_ref[...] = eff
    e_iota = jax.lax.broadcasted_iota(jnp.int32, (TB, HOPS, E), 2)
    oh = (eff[:, :, None] == e_iota).astype(jnp.float32)
    c = jnp.sum(oh, axis=0)

    @pl.when(i == 0)
    def _():
        acc[...] = jnp.zeros_like(acc)

    acc[...] += c

    @pl.when(i == NTB - 1)
    def _():
        cnt_ref[...] = acc[...].astype(jnp.int32)


def _stage_a(x, Wp, bp2, Ws2, Wrz, Wrs):
    return pl.pallas_call(
        _stage_a_body,
        grid=(NTB,),
        in_specs=[
            pl.BlockSpec((TB, D), lambda i: (i, 0)),
            pl.BlockSpec((D, D), lambda i: (0, 0)),
            pl.BlockSpec((1, D), lambda i: (0, 0)),
            pl.BlockSpec((D, E * SYM), lambda i: (0, 0)),
            pl.BlockSpec((D, HOPS * (E + 1)), lambda i: (0, 0)),
            pl.BlockSpec((SYM, HOPS * (E + 1)), lambda i: (0, 0)),
        ],
        out_specs=[
            pl.BlockSpec((TB, D), lambda i: (i, 0)),
            pl.BlockSpec((TB, E * SYM), lambda i: (i, 0)),
            pl.BlockSpec((TB, HOPS), lambda i: (i, 0)),
            pl.BlockSpec((TB, HOPS), lambda i: (i, 0)),
            pl.BlockSpec((HOPS, E), lambda i: (0, 0)),
        ],
        out_shape=[
            jax.ShapeDtypeStruct((B, D), jnp.float32),
            jax.ShapeDtypeStruct((B, E * SYM), jnp.float32),
            jax.ShapeDtypeStruct((B, HOPS), jnp.int32),
            jax.ShapeDtypeStruct((B, HOPS), jnp.int32),
            jax.ShapeDtypeStruct((HOPS, E), jnp.int32),
        ],
        scratch_shapes=[pltpu.VMEM((HOPS, E), jnp.float32)],
    )(x, Wp, bp2, Ws2, Wrz, Wrs)


def _sched_body(cnt_ref, base_ref, be_ref):
    c = cnt_ref[...].astype(jnp.float32)
    nb = jnp.ceil(c / TB)
    e1 = jax.lax.broadcasted_iota(jnp.int32, (E, E), 0)
    e2 = jax.lax.broadcasted_iota(jnp.int32, (E, E), 1)
    mexc = (e1 < e2).astype(jnp.float32)
    minc = (e1 <= e2).astype(jnp.float32)
    excl = jnp.dot(nb, mexc, preferred_element_type=jnp.float32)
    incl = jnp.dot(nb, minc, preferred_element_type=jnp.float32)
    base_ref[...] = (TB * excl).astype(jnp.int32)
    g_iota = jax.lax.broadcasted_iota(jnp.int32, (HOPS, NG, E), 1)
    be = jnp.sum((incl.astype(jnp.int32)[:, None, :] <= g_iota).astype(jnp.int32),
                 axis=2)
    tot = incl.astype(jnp.int32)[:, E - 1:E]
    g2 = jax.lax.broadcasted_iota(jnp.int32, (HOPS, NG), 1)
    be_ref[...] = jnp.where(g2 < tot, jnp.minimum(be, E - 1), -1)


def _sched(cnt):
    return pl.pallas_call(
        _sched_body,
        out_shape=[
            jax.ShapeDtypeStruct((HOPS, E), jnp.int32),
            jax.ShapeDtypeStruct((HOPS, NG), jnp.int32),
        ],
    )(cnt)


def _pos_body(eff_ref, base_ref, pos_ref, fin_ref, carry):
    i = pl.program_id(0)

    @pl.when(i == 0)
    def _():
        carry[...] = jnp.zeros_like(carry)

    eff = eff_ref[...]
    base = base_ref[...].astype(jnp.float32)
    run = carry[...]
    r1 = jax.lax.broadcasted_iota(jnp.int32, (TB, TB), 0)
    c1 = jax.lax.broadcasted_iota(jnp.int32, (TB, TB), 1)
    ltri = (c1 < r1).astype(jnp.float32)
    gtok = (i * TB + jax.lax.broadcasted_iota(jnp.int32, (TB, 1), 0)
            ).astype(jnp.float32)
    fin = HOPS * G + gtok
    pcols, newrun = [], []
    for h in range(HOPS):
        e_iota = jax.lax.broadcasted_iota(jnp.int32, (TB, E), 1)
        oh = (eff[:, h:h + 1] == e_iota).astype(jnp.float32)
        rank = jnp.dot(ltri, oh, preferred_element_type=jnp.float32)
        offs = base[h:h + 1, :] + run[h:h + 1, :]
        p = jnp.sum(oh * (rank + offs), axis=1, keepdims=True)
        is_act = jnp.sum(oh, axis=1, keepdims=True) > 0.0
        pcols.append(jnp.where(is_act, p, -1.0))
        fin = jnp.where(is_act, h * G + p, fin)
        newrun.append(run[h:h + 1, :] + jnp.sum(oh, axis=0, keepdims=True))
    pos_ref[...] = jnp.concatenate(pcols, axis=1).astype(jnp.int32)
    fin_ref[...] = fin.astype(jnp.int32)
    carry[...] = jnp.concatenate(newrun, axis=0)


def _pos(eff, base):
    return pl.pallas_call(
        _pos_body,
        grid=(NTB,),
        in_specs=[
            pl.BlockSpec((TB, HOPS), lambda i: (i, 0)),
            pl.BlockSpec((HOPS, E), lambda i: (0, 0)),
        ],
        out_specs=[
            pl.BlockSpec((TB, HOPS), lambda i: (i, 0)),
            pl.BlockSpec((TB, 1), lambda i: (i, 0)),
        ],
        out_shape=[
            jax.ShapeDtypeStruct((B, HOPS), jnp.int32),
            jax.ShapeDtypeStruct((B, 1), jnp.int32),
        ],
        scratch_shapes=[pltpu.VMEM((HOPS, E), jnp.float32)],
    )(eff, base)


def _cids_body(pos_ref, out_ref):
    g = pl.program_id(0)
    pos = pos_ref[...]
    tok = jax.lax.broadcasted_iota(jnp.int32, (B, TB), 0)
    slot = g * TB + jax.lax.broadcasted_iota(jnp.int32, (B, TB), 1)
    big = jnp.int32(1 << 30)
    rows = []
    for h in range(HOPS):
        p = pos[:, h:h + 1]
        if h == 0:
            src = tok
        else:
            src = (h - 1) * G + pos[:, h - 1:h] + jnp.zeros_like(tok)
        cand = jnp.where(p == slot, src, big)
        row = jnp.min(cand, axis=0, keepdims=True)
        rows.append(jnp.where(row == big, 0, row))
    out_ref[...] = jnp.concatenate(rows, axis=0)


def _cids(pos):
    return pl.pallas_call(
        _cids_body,
        grid=(NG,),
        in_specs=[pl.BlockSpec((B, HOPS), lambda g: (0, 0))],
        out_specs=pl.BlockSpec((HOPS, TB), lambda g: (0, g)),
        out_shape=jax.ShapeDtypeStruct((HOPS, G), jnp.int32),
    )(pos)


def _expert0_body(be_ref, g_ref, z_ref, wo_ref, bo_ref, o_ref):
    g = pl.program_id(0)

    @pl.when((g < NG) & (be_ref[jnp.minimum(g, NG - 1)] >= 0))
    def _():
        o_ref[...] = jnp.tanh(
            jnp.dot(g_ref[...], wo_ref[0], preferred_element_type=jnp.float32)
            + bo_ref[0])

    @pl.when(g >= NG)
    def _():
        o_ref[...] = z_ref[...]


def _expert0(be, gathered, z, Wo, bo3):
    grid_spec = pltpu.PrefetchScalarGridSpec(
        num_scalar_prefetch=1,
        grid=(NGX,),
        in_specs=[
            pl.BlockSpec((TB, D), lambda g, be: (jnp.minimum(g, NG - 1), 0)),
            pl.BlockSpec((TB, D), lambda g, be: (jnp.maximum(g - NG, 0), 0)),
            pl.BlockSpec((1, D, D),
                         lambda g, be: (jnp.maximum(be[jnp.minimum(g, NG - 1)], 0),
                                        0, 0)),
            pl.BlockSpec((1, 1, D),
                         lambda g, be: (jnp.maximum(be[jnp.minimum(g, NG - 1)], 0),
                                        0, 0)),
        ],
        out_specs=pl.BlockSpec(
            (TB, D),
            lambda g, be: (jnp.where(g < NG, g, HOPS * NG + g - NG), 0)),
    )
    return pl.pallas_call(
        _expert0_body,
        grid_spec=grid_spec,
        out_shape=jax.ShapeDtypeStruct((BIGN, D), jnp.float32),
    )(be, gathered, z, Wo, bo3)


def _expert_acc_body(be_ref, g_ref, wo_ref, bo_ref, acc_ref, o_ref):
    g = pl.program_id(0)

    @pl.when(be_ref[g] >= 0)
    def _():
        o_ref[...] = jnp.tanh(
            jnp.dot(g_ref[...], wo_ref[0], preferred_element_type=jnp.float32)
            + bo_ref[0])


def _expert_acc(h, be, gathered, Wo, bo3, bigacc):
    grid_spec = pltpu.PrefetchScalarGridSpec(
        num_scalar_prefetch=1,
        grid=(NG,),
        in_specs=[
            pl.BlockSpec((TB, D), lambda g, be: (g, 0)),
            pl.BlockSpec((1, D, D),
                         lambda g, be: (jnp.maximum(be[g], 0), 0, 0)),
            pl.BlockSpec((1, 1, D),
                         lambda g, be: (jnp.maximum(be[g], 0), 0, 0)),
            pl.BlockSpec(memory_space=pltpu.MemorySpace.HBM),
        ],
        out_specs=pl.BlockSpec((TB, D), lambda g, be, _h=h: (_h * NG + g, 0)),
    )
    return pl.pallas_call(
        _expert_acc_body,
        grid_spec=grid_spec,
        out_shape=jax.ShapeDtypeStruct((BIGN, D), jnp.float32),
        input_output_aliases={4: 0},
    )(be, gathered, Wo, bo3, bigacc)


def _sc_mesh():
    return plsc.VectorSubcoreMesh(core_axis_name="c", subcore_axis_name="s")


def _gather_sc(src, idx3, nch, ch):
    n_out = NW * nch * ch

    @functools.partial(
        pl.kernel,
        mesh=_sc_mesh(),
        out_type=jax.ShapeDtypeStruct((n_out, D), jnp.float32),
        scratch_types=[
            pltpu.VMEM((nch, ch), jnp.int32),
            pltpu.VMEM((ch, D), jnp.float32),
            pltpu.VMEM((ch, D), jnp.float32),
            pltpu.SemaphoreType.DMA,
            pltpu.SemaphoreType.DMA,
            pltpu.SemaphoreType.DMA,
        ],
    )
    def k(src_hbm, idx_hbm, out_hbm, idx_v, buf0, buf1, sg0, sg1, sw):
        wid = lax.axis_index("s") * 2 + lax.axis_index("c")
        base = wid * nch * ch
        pltpu.sync_copy(idx_hbm.at[wid], idx_v)
        bufs = (buf0, buf1)
        sgs = (sg0, sg1)
        pltpu.make_async_copy(src_hbm.at[idx_v.at[0]], buf0, sg0).start()
        for j in range(nch):
            cur, sc = bufs[j % 2], sgs[j % 2]
            pltpu.make_async_copy(src_hbm.at[idx_v.at[j]], cur, sc).wait()
            if j + 1 < nch:
                nxt, sn = bufs[(j + 1) % 2], sgs[(j + 1) % 2]
                if j >= 1:
                    pltpu.make_async_copy(nxt, out_hbm.at[pl.ds(0, ch)], sw).wait()
                pltpu.make_async_copy(src_hbm.at[idx_v.at[j + 1]], nxt, sn).start()
            pltpu.make_async_copy(cur, out_hbm.at[pl.ds(base + j * ch, ch)],
                                  sw).start()
        for _ in range(2 if nch >= 2 else 1):
            pltpu.make_async_copy(buf0, out_hbm.at[pl.ds(0, ch)], sw).wait()

    return k(src, idx3)


def kernel(x, Wp, bp, Wo, bo, Ws, Wr, max_ops):
    Ws2 = jnp.transpose(Ws, (1, 0, 2)).reshape(D, E * SYM)
    Wrz = jnp.transpose(Wr[:, :D, :], (1, 0, 2)).reshape(D, HOPS * (E + 1))
    Wrs = jnp.transpose(Wr[:, D:, :], (1, 0, 2)).reshape(SYM, HOPS * (E + 1))
    bp2 = bp.reshape(1, D)
    bo3 = bo.reshape(E, 1, D)

    z, sym_flat, prog, eff, cnt = _stage_a(x, Wp, bp2, Ws2, Wrz, Wrs)
    basearr, be = _sched(cnt)
    pos, fin = _pos(eff, basearr)
    cids = _cids(pos)

    gathered = _gather_sc(z, cids[0].reshape(NW, 4, 48), 4, 48)
    bigacc = _expert0(be[0], gathered, z, Wo, bo3)
    for h in range(1, HOPS):
        gathered = _gather_sc(bigacc, cids[h].reshape(NW, 4, 48), 4, 48)
        bigacc = _expert_acc(h, be[h], gathered, Wo, bo3, bigacc)
    out = _gather_sc(bigacc, fin.reshape(NW, 4, 32), 4, 32)

    return out, prog, sym_flat.reshape(B, E, SYM)
